# Initial kernel scaffold; baseline (speedup 1.0000x reference)
import jax
import jax.numpy as jnp
from jax.experimental import pallas as pl


def kernel(x, edge_index, edge_attr, batch, params):
    raise NotImplementedError("write your pallas kernel here")


# speedup vs baseline: 1.9687x; 1.9687x over previous
import functools

import jax
import jax.numpy as jnp
from jax import lax
from jax.experimental import pallas as pl
from jax.experimental.pallas import tpu as pltpu
from jax.experimental.pallas import tpu_sc as plsc

N = 10000
E = 160000
IN_DIM = 177
EDGE_DIM = 30
HID = 512
PROJ = 768
L = 6
B = 64
EPS = 1.1920928955078125e-07

NCHUNK = 4
CW = HID // NCHUNK
NTILE = 16
RPT = 624
RTAIL = N - NTILE * RPT
RTB = NTILE * RPT
EB = 80
EPT = E // NTILE
NJ = EPT // EB

_mesh = plsc.VectorSubcoreMesh(core_axis_name="c", subcore_axis_name="s",
                               num_cores=2, num_subcores=NTILE)


def _lin_body(x_ref, w_ref, b_ref, o_ref):
    o_ref[...] = (
        jnp.dot(x_ref[...], w_ref[...], preferred_element_type=jnp.float32)
        + b_ref[...]
    )


def _linear(x, w, b, bm):
    m, k = x.shape
    n = w.shape[1]
    return pl.pallas_call(
        _lin_body,
        grid=(m // bm,),
        in_specs=[
            pl.BlockSpec((bm, k), lambda i: (i, 0)),
            pl.BlockSpec((k, n), lambda i: (0, 0)),
            pl.BlockSpec((1, n), lambda i: (0, 0)),
        ],
        out_specs=pl.BlockSpec((bm, n), lambda i: (i, 0)),
        out_shape=jax.ShapeDtypeStruct((m, n), jnp.float32),
    )(x, w, b)


def _chunked_body(x_ref, w_ref, b_ref, o_ref):
    o_ref[...] = (
        jnp.dot(x_ref[...], w_ref[0], preferred_element_type=jnp.float32)
        + b_ref[0]
    )[None]


def _chunked_matmul(x, w, b, bm):
    m, k = x.shape
    return pl.pallas_call(
        _chunked_body,
        grid=(NCHUNK, m // bm),
        in_specs=[
            pl.BlockSpec((bm, k), lambda c, i: (i, 0)),
            pl.BlockSpec((1, k, CW), lambda c, i: (c, 0, 0)),
            pl.BlockSpec((1, 1, CW), lambda c, i: (c, 0, 0)),
        ],
        out_specs=pl.BlockSpec((1, bm, CW), lambda c, i: (c, i, 0)),
        out_shape=jax.ShapeDtypeStruct((NCHUNK, m, CW), jnp.float32),
    )(x, w, b)


def _rms(v, w):
    return v * lax.rsqrt(jnp.mean(v * v, axis=-1, keepdims=True) + EPS) * w


def _upd_body(h_ref, s_ref, deg_ref, wm2_ref, bm2_ref, wuh_ref, wua_ref,
              bu_ref, nw_ref, wf1_ref, bf1_ref, wf2_ref, bf2_ref, o_ref):
    h = h_ref[...]
    t = jnp.dot(s_ref[0], wm2_ref[0], preferred_element_type=jnp.float32)
    for c in range(1, NCHUNK):
        t = t + jnp.dot(s_ref[c], wm2_ref[c], preferred_element_type=jnp.float32)
    aggr = t + deg_ref[...] * bm2_ref[...]
    u = jnp.maximum(
        jnp.dot(h, wuh_ref[...], preferred_element_type=jnp.float32)
        + jnp.dot(aggr, wua_ref[...], preferred_element_type=jnp.float32)
        + bu_ref[...], 0.0)
    h1 = _rms(h + u, nw_ref[...])
    f = jnp.dot(
        jnp.maximum(jnp.dot(h1, wf1_ref[...], preferred_element_type=jnp.float32)
                    + bf1_ref[...], 0.0),
        wf2_ref[...], preferred_element_type=jnp.float32) + bf2_ref[...]
    o_ref[...] = _rms(h1 + f, nw_ref[...])


def _update(h, s4, deg, wm2c, bm2, wuh, wua, bu, nw, wf1, bf1, wf2, bf2, bm):
    full = lambda shape: pl.BlockSpec(shape, lambda i: tuple(0 for _ in shape))
    return pl.pallas_call(
        _upd_body,
        grid=(N // bm,),
        in_specs=[
            pl.BlockSpec((bm, HID), lambda i: (i, 0)),
            pl.BlockSpec((NCHUNK, bm, CW), lambda i: (0, i, 0)),
            pl.BlockSpec((bm, 1), lambda i: (i, 0)),
            full((NCHUNK, CW, HID)),
            full((1, HID)),
            full((HID, HID)),
            full((HID, HID)),
            full((1, HID)),
            full((1, HID)),
            full((HID, HID)),
            full((1, HID)),
            full((HID, HID)),
            full((1, HID)),
        ],
        out_specs=pl.BlockSpec((bm, HID), lambda i: (i, 0)),
        out_shape=jax.ShapeDtypeStruct((N, HID), jnp.float32),
    )(h, s4, deg, wm2c, bm2, wuh, wua, bu, nw, wf1, bf1, wf2, bf2)


def _pool1_body(h_ref, batch_ref, wg_ref, bg_ref, wa_ref, ba_ref,
                z_ref, att_ref, m_ref, cnt_ref):
    z = jnp.dot(h_ref[...], wg_ref[...], preferred_element_type=jnp.float32) + bg_ref[...]
    att = jnp.dot(z, wa_ref[...], preferred_element_type=jnp.float32) + ba_ref[...]
    z_ref[...] = z
    att_ref[...] = att
    iota = lax.broadcasted_iota(jnp.int32, (z.shape[0], B), 1)
    oh = batch_ref[...] == iota
    mloc = jnp.max(jnp.where(oh, att, -jnp.inf), axis=0, keepdims=True)
    cloc = jnp.sum(oh.astype(jnp.float32), axis=0, keepdims=True)

    @pl.when(pl.program_id(0) == 0)
    def _():
        m_ref[...] = jnp.full((1, B), -jnp.inf, jnp.float32)
        cnt_ref[...] = jnp.zeros((1, B), jnp.float32)

    m_ref[...] = jnp.maximum(m_ref[...], mloc)
    cnt_ref[...] = cnt_ref[...] + cloc


def _pool1(h, batch2, wg, bg, wa, ba, bm):
    full = lambda shape: pl.BlockSpec(shape, lambda i: tuple(0 for _ in shape))
    return pl.pallas_call(
        _pool1_body,
        grid=(N // bm,),
        in_specs=[
            pl.BlockSpec((bm, HID), lambda i: (i, 0)),
            pl.BlockSpec((bm, 1), lambda i: (i, 0)),
            full((HID, HID)),
            full((1, HID)),
            full((HID, 1)),
            full((1, 1)),
        ],
        out_specs=[
            pl.BlockSpec((bm, HID), lambda i: (i, 0)),
            pl.BlockSpec((bm, 1), lambda i: (i, 0)),
            full((1, B)),
            full((1, B)),
        ],
        out_shape=[
            jax.ShapeDtypeStruct((N, HID), jnp.float32),
            jax.ShapeDtypeStruct((N, 1), jnp.float32),
            jax.ShapeDtypeStruct((1, B), jnp.float32),
            jax.ShapeDtypeStruct((1, B), jnp.float32),
        ],
    )(h, batch2, wg, bg, wa, ba)


def _poolm_body(m_ref, cnt_ref, bg_ref, wa_ref, ba_ref,
                mfin_ref, pad_ref, apad_ref):
    apad = jnp.dot(bg_ref[...], wa_ref[...], preferred_element_type=jnp.float32) + ba_ref[...]
    cnt = cnt_ref[...]
    mmax = jnp.max(cnt)
    pad = mmax - cnt
    mfin_ref[...] = jnp.where(pad > 0, jnp.maximum(m_ref[...], apad), m_ref[...])
    pad_ref[...] = pad
    apad_ref[...] = apad


def _poolm(m_nodes, cnt, bg, wa, ba):
    full = lambda shape: pl.BlockSpec(shape, lambda: tuple(0 for _ in shape))
    return pl.pallas_call(
        _poolm_body,
        in_specs=[full((1, B)), full((1, B)), full((1, HID)), full((HID, 1)),
                  full((1, 1))],
        out_specs=[full((1, B)), full((1, B)), full((1, 1))],
        out_shape=[
            jax.ShapeDtypeStruct((1, B), jnp.float32),
            jax.ShapeDtypeStruct((1, B), jnp.float32),
            jax.ShapeDtypeStruct((1, 1), jnp.float32),
        ],
    )(m_nodes, cnt, bg, wa, ba)


def _pool2_body(z_ref, att_ref, batch_ref, mfin_ref, sz_ref, s1_ref):
    bm = z_ref.shape[0]
    iota = lax.broadcasted_iota(jnp.int32, (bm, B), 1)
    oh = (batch_ref[...] == iota).astype(jnp.float32)
    msel = jnp.sum(oh * mfin_ref[...], axis=1, keepdims=True)
    w = jnp.exp(att_ref[...] - msel)
    ow = oh * w
    szloc = lax.dot_general(ow, z_ref[...], (((0,), (0,)), ((), ())),
                            preferred_element_type=jnp.float32)
    s1loc = jnp.sum(ow, axis=0, keepdims=True)

    @pl.when(pl.program_id(0) == 0)
    def _():
        sz_ref[...] = jnp.zeros_like(sz_ref)
        s1_ref[...] = jnp.zeros_like(s1_ref)

    sz_ref[...] = sz_ref[...] + szloc
    s1_ref[...] = s1_ref[...] + s1loc


def _pool2(z, att, batch2, mfin, bm):
    full = lambda shape: pl.BlockSpec(shape, lambda i: tuple(0 for _ in shape))
    return pl.pallas_call(
        _pool2_body,
        grid=(N // bm,),
        in_specs=[
            pl.BlockSpec((bm, HID), lambda i: (i, 0)),
            pl.BlockSpec((bm, 1), lambda i: (i, 0)),
            pl.BlockSpec((bm, 1), lambda i: (i, 0)),
            full((1, B)),
        ],
        out_specs=[full((B, HID)), full((1, B))],
        out_shape=[
            jax.ShapeDtypeStruct((B, HID), jnp.float32),
            jax.ShapeDtypeStruct((1, B), jnp.float32),
        ],
    )(z, att, batch2, mfin)


def _pool3_body(sz_ref, s1_ref, mfin_ref, pad_ref, apad_ref,
                wp1_ref, bp1_ref, pnw_ref, wp2_ref, bp2_ref, o_ref):
    denom = s1_ref[...] + pad_ref[...] * jnp.exp(apad_ref[...] - mfin_ref[...])
    hg = sz_ref[...] / denom
    p = jnp.dot(hg, wp1_ref[...], preferred_element_type=jnp.float32) + bp1_ref[...]
    p = jnp.maximum(_rms(p, pnw_ref[...]), 0.0)
    o_ref[...] = jnp.dot(p, wp2_ref[...], preferred_element_type=jnp.float32) + bp2_ref[...]


def _pool3(sz, s1c, mfinc, padc, apad, wp1, bp1, pnw, wp2, bp2):
    full = lambda shape: pl.BlockSpec(shape, lambda: tuple(0 for _ in shape))
    return pl.pallas_call(
        _pool3_body,
        in_specs=[full((B, HID)), full((B, 1)), full((B, 1)), full((B, 1)),
                  full((1, 1)), full((HID, PROJ)), full((1, PROJ)),
                  full((1, PROJ)), full((PROJ, PROJ)), full((1, PROJ))],
        out_specs=full((B, PROJ)),
        out_shape=jax.ShapeDtypeStruct((B, PROJ), jnp.float32),
    )(sz, s1c, mfinc, padc, apad, wp1, bp1, pnw, wp2, bp2)


@functools.partial(
    pl.kernel,
    out_type=jax.ShapeDtypeStruct((N, 16), jnp.float32),
    mesh=_mesh,
    scratch_types=[
        pltpu.VMEM((EB,), jnp.int32),
        pltpu.VMEM((EB, 16), jnp.float32),
        pltpu.VMEM_SHARED((N, 16), jnp.float32),
    ],
)
def _deg_kernel(dst_hbm, zeros_hbm, out_hbm, idx_v, ones_v, acc):
    cid = lax.axis_index("c")
    tid = lax.axis_index("s")

    @pl.when(cid == 0)
    def _():
        def fill(i, _):
            ones_v[i, :] = jnp.full((16,), 1.0, jnp.float32)
            return 0
        lax.fori_loop(0, EB, fill, 0)
        pltpu.sync_copy(zeros_hbm.at[pl.ds(tid * RPT, RPT)],
                        acc.at[pl.ds(tid * RPT, RPT)])

        @pl.when(tid == 0)
        def _():
            pltpu.sync_copy(zeros_hbm.at[pl.ds(RTB, RTAIL)],
                            acc.at[pl.ds(RTB, RTAIL)])
        plsc.subcore_barrier()
        base = tid * EPT

        def step(j, _):
            pltpu.sync_copy(dst_hbm.at[pl.ds(base + j * EB, EB)], idx_v)
            pltpu.sync_copy(ones_v, acc.at[idx_v], add=True)
            return 0
        lax.fori_loop(0, NJ, step, 0)
        plsc.subcore_barrier()
        pltpu.sync_copy(acc.at[pl.ds(tid * RPT, RPT)],
                        out_hbm.at[pl.ds(tid * RPT, RPT)])

        @pl.when(tid == 0)
        def _():
            pltpu.sync_copy(acc.at[pl.ds(RTB, RTAIL)],
                            out_hbm.at[pl.ds(RTB, RTAIL)])


@functools.partial(
    pl.kernel,
    out_type=jax.ShapeDtypeStruct((NCHUNK * N, CW), jnp.float32),
    mesh=_mesh,
    scratch_types=[
        pltpu.VMEM((EB,), jnp.int32),
        pltpu.VMEM((EB,), jnp.int32),
        pltpu.VMEM((EB, CW), jnp.float32),
        pltpu.VMEM((EB, CW), jnp.float32),
        pltpu.VMEM_SHARED((N, CW), jnp.float32),
        pltpu.SemaphoreType.DMA,
    ],
)
def _edge_kernel(hm_hbm, srcadj_hbm, dst_hbm, em_hbm, zeros_hbm, out_hbm,
                 sidx, didx, gbuf, ebuf, acc, gsem):
    cid = lax.axis_index("c")
    tid = lax.axis_index("s")
    for cc in range(NCHUNK // 2):
        chunk = cid * (NCHUNK // 2) + cc
        pltpu.sync_copy(zeros_hbm.at[pl.ds(tid * RPT, RPT)],
                        acc.at[pl.ds(tid * RPT, RPT)])

        @pl.when(tid == 0)
        def _():
            pltpu.sync_copy(zeros_hbm.at[pl.ds(RTB, RTAIL)],
                            acc.at[pl.ds(RTB, RTAIL)])
        plsc.subcore_barrier()
        ebase = chunk * E + tid * EPT
        dbase = tid * EPT

        def step(j, _):
            off = j * EB
            pltpu.sync_copy(srcadj_hbm.at[pl.ds(ebase + off, EB)], sidx)
            pltpu.sync_copy(dst_hbm.at[pl.ds(dbase + off, EB)], didx)
            pltpu.sync_copy(em_hbm.at[pl.ds(ebase + off, EB)], ebuf)
            pltpu.async_copy(hm_hbm.at[sidx], gbuf, gsem).wait()

            def crow(r, _2):
                for kk in range(CW // 16):
                    sl = pl.ds(kk * 16, 16)
                    gbuf[r, sl] = jnp.maximum(gbuf[r, sl] + ebuf[r, sl], 0.0)
                return 0
            lax.fori_loop(0, EB, crow, 0)
            pltpu.sync_copy(gbuf, acc.at[didx], add=True)
            return 0
        lax.fori_loop(0, NJ, step, 0)
        plsc.subcore_barrier()
        pltpu.sync_copy(acc.at[pl.ds(tid * RPT, RPT)],
                        out_hbm.at[pl.ds(chunk * N + tid * RPT, RPT)])

        @pl.when(tid == 0)
        def _():
            pltpu.sync_copy(acc.at[pl.ds(RTB, RTAIL)],
                            out_hbm.at[pl.ds(chunk * N + RTB, RTAIL)])


def kernel(x, edge_index, edge_attr, batch, params):
    p = params
    src = edge_index[0]
    dst = edge_index[1]
    batch2 = batch[:, None]

    zeros_cw = jnp.zeros((N, CW), jnp.float32)
    zeros_16 = jnp.zeros((N, 16), jnp.float32)
    srcadj = (src[None, :]
              + (N * jnp.arange(NCHUNK, dtype=jnp.int32))[:, None]).reshape(-1)

    h = _linear(x, p['W_in'], p['b_in'][None], 1000)
    deg = _deg_kernel(dst, zeros_16)[:, 0:1]

    for i in range(L):
        wm1 = p['Wm1'][i]
        wm1h = wm1[:HID].reshape(HID, NCHUNK, CW).transpose(1, 0, 2)
        wm1e = wm1[HID:].reshape(EDGE_DIM, NCHUNK, CW).transpose(1, 0, 2)
        bm1 = p['bm1'][i].reshape(NCHUNK, 1, CW)
        zero_b = jnp.zeros((NCHUNK, 1, CW), jnp.float32)
        hm = _chunked_matmul(h, wm1h, bm1, 1000)
        em = _chunked_matmul(edge_attr, wm1e, zero_b, 2000)
        s = _edge_kernel(hm.reshape(NCHUNK * N, CW), srcadj, dst,
                         em.reshape(NCHUNK * E, CW), zeros_cw)
        s4 = s.reshape(NCHUNK, N, CW)
        h = _update(h, s4, deg,
                    p['Wm2'][i].reshape(NCHUNK, CW, HID), p['bm2'][i][None],
                    p['Wu'][i][:HID], p['Wu'][i][HID:], p['bu'][i][None],
                    p['norm_w'][i][None],
                    p['Wf1'][i], p['bf1'][i][None],
                    p['Wf2'][i], p['bf2'][i][None], 1000)

    z, att, m_nodes, cnt = _pool1(h, batch2, p['Wg'], p['bg'][None],
                                  p['Wa'], p['ba'][None], 1000)
    mfin, pad, apad = _poolm(m_nodes, cnt, p['bg'][None], p['Wa'], p['ba'][None])
    sz, s1 = _pool2(z, att, batch2, mfin, 1000)
    return _pool3(sz, s1.T, mfin.T, pad.T, apad,
                  p['Wp1'], p['bp1'][None], p['pnorm_w'][None],
                  p['Wp2'], p['bp2'][None])


# speedup vs baseline: 3.2049x; 1.6279x over previous
import functools

import jax
import jax.numpy as jnp
from jax import lax
from jax.experimental import pallas as pl
from jax.experimental.pallas import tpu as pltpu
from jax.experimental.pallas import tpu_sc as plsc

N = 10000
E = 160000
IN_DIM = 177
EDGE_DIM = 30
HID = 512
PROJ = 768
L = 6
B = 64
EPS = 1.1920928955078125e-07

NCHUNK = 4
CW = HID // NCHUNK
NTILE = 16
RPT = 624
RTAIL = N - NTILE * RPT
RTB = NTILE * RPT
EB = 80
EPT = E // NTILE
NJ = EPT // EB

_mesh = plsc.VectorSubcoreMesh(core_axis_name="c", subcore_axis_name="s",
                               num_cores=2, num_subcores=NTILE)


def _lin_body(x_ref, w_ref, b_ref, o_ref):
    o_ref[...] = (
        jnp.dot(x_ref[...], w_ref[...], preferred_element_type=jnp.float32)
        + b_ref[...]
    )


def _linear(x, w, b, bm):
    m, k = x.shape
    n = w.shape[1]
    return pl.pallas_call(
        _lin_body,
        grid=(m // bm,),
        in_specs=[
            pl.BlockSpec((bm, k), lambda i: (i, 0)),
            pl.BlockSpec((k, n), lambda i: (0, 0)),
            pl.BlockSpec((1, n), lambda i: (0, 0)),
        ],
        out_specs=pl.BlockSpec((bm, n), lambda i: (i, 0)),
        out_shape=jax.ShapeDtypeStruct((m, n), jnp.float32),
    )(x, w, b)


def _chunked_body(x_ref, w_ref, b_ref, o_ref):
    o_ref[...] = (
        jnp.dot(x_ref[...], w_ref[0], preferred_element_type=jnp.float32)
        + b_ref[0]
    )[None]


def _chunked_matmul(x, w, b, bm):
    m, k = x.shape
    return pl.pallas_call(
        _chunked_body,
        grid=(NCHUNK, m // bm),
        in_specs=[
            pl.BlockSpec((bm, k), lambda c, i: (i, 0)),
            pl.BlockSpec((1, k, CW), lambda c, i: (c, 0, 0)),
            pl.BlockSpec((1, 1, CW), lambda c, i: (c, 0, 0)),
        ],
        out_specs=pl.BlockSpec((1, bm, CW), lambda c, i: (c, i, 0)),
        out_shape=jax.ShapeDtypeStruct((NCHUNK, m, CW), jnp.float32),
    )(x, w, b)


def _rms(v, w):
    return v * lax.rsqrt(jnp.mean(v * v, axis=-1, keepdims=True) + EPS) * w


def _upd_body(h_ref, s_ref, deg_ref, wm2_ref, bm2_ref, wuh_ref, wua_ref,
              bu_ref, nw_ref, wf1_ref, bf1_ref, wf2_ref, bf2_ref, o_ref):
    h = h_ref[...]
    t = jnp.dot(s_ref[0], wm2_ref[0], preferred_element_type=jnp.float32)
    for c in range(1, NCHUNK):
        t = t + jnp.dot(s_ref[c], wm2_ref[c], preferred_element_type=jnp.float32)
    aggr = t + deg_ref[...] * bm2_ref[...]
    u = jnp.maximum(
        jnp.dot(h, wuh_ref[...], preferred_element_type=jnp.float32)
        + jnp.dot(aggr, wua_ref[...], preferred_element_type=jnp.float32)
        + bu_ref[...], 0.0)
    h1 = _rms(h + u, nw_ref[...])
    f = jnp.dot(
        jnp.maximum(jnp.dot(h1, wf1_ref[...], preferred_element_type=jnp.float32)
                    + bf1_ref[...], 0.0),
        wf2_ref[...], preferred_element_type=jnp.float32) + bf2_ref[...]
    o_ref[...] = _rms(h1 + f, nw_ref[...])


def _update(h, s4, deg, wm2c, bm2, wuh, wua, bu, nw, wf1, bf1, wf2, bf2, bm):
    full = lambda shape: pl.BlockSpec(shape, lambda i: tuple(0 for _ in shape))
    return pl.pallas_call(
        _upd_body,
        grid=(N // bm,),
        in_specs=[
            pl.BlockSpec((bm, HID), lambda i: (i, 0)),
            pl.BlockSpec((NCHUNK, bm, CW), lambda i: (0, i, 0)),
            pl.BlockSpec((bm, 1), lambda i: (i, 0)),
            full((NCHUNK, CW, HID)),
            full((1, HID)),
            full((HID, HID)),
            full((HID, HID)),
            full((1, HID)),
            full((1, HID)),
            full((HID, HID)),
            full((1, HID)),
            full((HID, HID)),
            full((1, HID)),
        ],
        out_specs=pl.BlockSpec((bm, HID), lambda i: (i, 0)),
        out_shape=jax.ShapeDtypeStruct((N, HID), jnp.float32),
    )(h, s4, deg, wm2c, bm2, wuh, wua, bu, nw, wf1, bf1, wf2, bf2)


def _pool1_body(h_ref, batch_ref, wg_ref, bg_ref, wa_ref, ba_ref,
                z_ref, att_ref, m_ref, cnt_ref):
    z = jnp.dot(h_ref[...], wg_ref[...], preferred_element_type=jnp.float32) + bg_ref[...]
    att = jnp.dot(z, wa_ref[...], preferred_element_type=jnp.float32) + ba_ref[...]
    z_ref[...] = z
    att_ref[...] = att
    iota = lax.broadcasted_iota(jnp.int32, (z.shape[0], B), 1)
    oh = batch_ref[...] == iota
    mloc = jnp.max(jnp.where(oh, att, -jnp.inf), axis=0, keepdims=True)
    cloc = jnp.sum(oh.astype(jnp.float32), axis=0, keepdims=True)

    @pl.when(pl.program_id(0) == 0)
    def _():
        m_ref[...] = jnp.full((1, B), -jnp.inf, jnp.float32)
        cnt_ref[...] = jnp.zeros((1, B), jnp.float32)

    m_ref[...] = jnp.maximum(m_ref[...], mloc)
    cnt_ref[...] = cnt_ref[...] + cloc


def _pool1(h, batch2, wg, bg, wa, ba, bm):
    full = lambda shape: pl.BlockSpec(shape, lambda i: tuple(0 for _ in shape))
    return pl.pallas_call(
        _pool1_body,
        grid=(N // bm,),
        in_specs=[
            pl.BlockSpec((bm, HID), lambda i: (i, 0)),
            pl.BlockSpec((bm, 1), lambda i: (i, 0)),
            full((HID, HID)),
            full((1, HID)),
            full((HID, 1)),
            full((1, 1)),
        ],
        out_specs=[
            pl.BlockSpec((bm, HID), lambda i: (i, 0)),
            pl.BlockSpec((bm, 1), lambda i: (i, 0)),
            full((1, B)),
            full((1, B)),
        ],
        out_shape=[
            jax.ShapeDtypeStruct((N, HID), jnp.float32),
            jax.ShapeDtypeStruct((N, 1), jnp.float32),
            jax.ShapeDtypeStruct((1, B), jnp.float32),
            jax.ShapeDtypeStruct((1, B), jnp.float32),
        ],
    )(h, batch2, wg, bg, wa, ba)


def _poolm_body(m_ref, cnt_ref, bg_ref, wa_ref, ba_ref,
                mfin_ref, pad_ref, apad_ref):
    apad = jnp.dot(bg_ref[...], wa_ref[...], preferred_element_type=jnp.float32) + ba_ref[...]
    cnt = cnt_ref[...]
    mmax = jnp.max(cnt)
    pad = mmax - cnt
    mfin_ref[...] = jnp.where(pad > 0, jnp.maximum(m_ref[...], apad), m_ref[...])
    pad_ref[...] = pad
    apad_ref[...] = apad


def _poolm(m_nodes, cnt, bg, wa, ba):
    full = lambda shape: pl.BlockSpec(shape, lambda: tuple(0 for _ in shape))
    return pl.pallas_call(
        _poolm_body,
        in_specs=[full((1, B)), full((1, B)), full((1, HID)), full((HID, 1)),
                  full((1, 1))],
        out_specs=[full((1, B)), full((1, B)), full((1, 1))],
        out_shape=[
            jax.ShapeDtypeStruct((1, B), jnp.float32),
            jax.ShapeDtypeStruct((1, B), jnp.float32),
            jax.ShapeDtypeStruct((1, 1), jnp.float32),
        ],
    )(m_nodes, cnt, bg, wa, ba)


def _pool2_body(z_ref, att_ref, batch_ref, mfin_ref, sz_ref, s1_ref):
    bm = z_ref.shape[0]
    iota = lax.broadcasted_iota(jnp.int32, (bm, B), 1)
    oh = (batch_ref[...] == iota).astype(jnp.float32)
    msel = jnp.sum(oh * mfin_ref[...], axis=1, keepdims=True)
    w = jnp.exp(att_ref[...] - msel)
    ow = oh * w
    szloc = lax.dot_general(ow, z_ref[...], (((0,), (0,)), ((), ())),
                            preferred_element_type=jnp.float32)
    s1loc = jnp.sum(ow, axis=0, keepdims=True)

    @pl.when(pl.program_id(0) == 0)
    def _():
        sz_ref[...] = jnp.zeros_like(sz_ref)
        s1_ref[...] = jnp.zeros_like(s1_ref)

    sz_ref[...] = sz_ref[...] + szloc
    s1_ref[...] = s1_ref[...] + s1loc


def _pool2(z, att, batch2, mfin, bm):
    full = lambda shape: pl.BlockSpec(shape, lambda i: tuple(0 for _ in shape))
    return pl.pallas_call(
        _pool2_body,
        grid=(N // bm,),
        in_specs=[
            pl.BlockSpec((bm, HID), lambda i: (i, 0)),
            pl.BlockSpec((bm, 1), lambda i: (i, 0)),
            pl.BlockSpec((bm, 1), lambda i: (i, 0)),
            full((1, B)),
        ],
        out_specs=[full((B, HID)), full((1, B))],
        out_shape=[
            jax.ShapeDtypeStruct((B, HID), jnp.float32),
            jax.ShapeDtypeStruct((1, B), jnp.float32),
        ],
    )(z, att, batch2, mfin)


def _pool3_body(sz_ref, s1_ref, mfin_ref, pad_ref, apad_ref,
                wp1_ref, bp1_ref, pnw_ref, wp2_ref, bp2_ref, o_ref):
    denom = s1_ref[...] + pad_ref[...] * jnp.exp(apad_ref[...] - mfin_ref[...])
    hg = sz_ref[...] / denom
    p = jnp.dot(hg, wp1_ref[...], preferred_element_type=jnp.float32) + bp1_ref[...]
    p = jnp.maximum(_rms(p, pnw_ref[...]), 0.0)
    o_ref[...] = jnp.dot(p, wp2_ref[...], preferred_element_type=jnp.float32) + bp2_ref[...]


def _pool3(sz, s1c, mfinc, padc, apad, wp1, bp1, pnw, wp2, bp2):
    full = lambda shape: pl.BlockSpec(shape, lambda: tuple(0 for _ in shape))
    return pl.pallas_call(
        _pool3_body,
        in_specs=[full((B, HID)), full((B, 1)), full((B, 1)), full((B, 1)),
                  full((1, 1)), full((HID, PROJ)), full((1, PROJ)),
                  full((1, PROJ)), full((PROJ, PROJ)), full((1, PROJ))],
        out_specs=full((B, PROJ)),
        out_shape=jax.ShapeDtypeStruct((B, PROJ), jnp.float32),
    )(sz, s1c, mfinc, padc, apad, wp1, bp1, pnw, wp2, bp2)


@functools.partial(
    pl.kernel,
    out_type=jax.ShapeDtypeStruct((N, 16), jnp.float32),
    mesh=_mesh,
    scratch_types=[
        pltpu.VMEM((EB,), jnp.int32),
        pltpu.VMEM((EB, 16), jnp.float32),
        pltpu.VMEM_SHARED((N, 16), jnp.float32),
    ],
)
def _deg_kernel(dst_hbm, zeros_hbm, out_hbm, idx_v, ones_v, acc):
    cid = lax.axis_index("c")
    tid = lax.axis_index("s")

    @pl.when(cid == 0)
    def _():
        def fill(i, _):
            ones_v[i, :] = jnp.full((16,), 1.0, jnp.float32)
            return 0
        lax.fori_loop(0, EB, fill, 0)
        pltpu.sync_copy(zeros_hbm.at[pl.ds(tid * RPT, RPT)],
                        acc.at[pl.ds(tid * RPT, RPT)])

        @pl.when(tid == 0)
        def _():
            pltpu.sync_copy(zeros_hbm.at[pl.ds(RTB, RTAIL)],
                            acc.at[pl.ds(RTB, RTAIL)])
        plsc.subcore_barrier()
        base = tid * EPT

        def step(j, _):
            pltpu.sync_copy(dst_hbm.at[pl.ds(base + j * EB, EB)], idx_v)
            pltpu.sync_copy(ones_v, acc.at[idx_v], add=True)
            return 0
        lax.fori_loop(0, NJ, step, 0)
        plsc.subcore_barrier()
        pltpu.sync_copy(acc.at[pl.ds(tid * RPT, RPT)],
                        out_hbm.at[pl.ds(tid * RPT, RPT)])

        @pl.when(tid == 0)
        def _():
            pltpu.sync_copy(acc.at[pl.ds(RTB, RTAIL)],
                            out_hbm.at[pl.ds(RTB, RTAIL)])


@functools.partial(
    pl.kernel,
    out_type=jax.ShapeDtypeStruct((NCHUNK * N, CW), jnp.float32),
    mesh=_mesh,
    scratch_types=[
        pltpu.VMEM((EB,), jnp.int32),
        pltpu.VMEM((EB,), jnp.int32),
        pltpu.VMEM((EB,), jnp.int32),
        pltpu.VMEM((EB,), jnp.int32),
        pltpu.VMEM((EB,), jnp.int32),
        pltpu.VMEM((EB,), jnp.int32),
        pltpu.VMEM((EB,), jnp.int32),
        pltpu.VMEM((EB,), jnp.int32),
        pltpu.VMEM((EB, CW), jnp.float32),
        pltpu.VMEM((EB, CW), jnp.float32),
        pltpu.VMEM((EB, CW), jnp.float32),
        pltpu.VMEM((EB, CW), jnp.float32),
        pltpu.VMEM_SHARED((N, CW), jnp.float32),
        pltpu.SemaphoreType.DMA,
        pltpu.SemaphoreType.DMA,
        pltpu.SemaphoreType.DMA,
        pltpu.SemaphoreType.DMA,
        pltpu.SemaphoreType.DMA,
        pltpu.SemaphoreType.DMA,
        pltpu.SemaphoreType.DMA,
        pltpu.SemaphoreType.DMA,
        pltpu.SemaphoreType.DMA,
        pltpu.SemaphoreType.DMA,
    ],
)
def _edge_kernel(hm_hbm, srcadj_hbm, dst_hbm, em_hbm, zeros_hbm, out_hbm,
                 si0, si1, si2, si3, di0, di1, di2, di3,
                 eb0, eb1, gb0, gb1, acc,
                 xs0, xs1, xs2, xs3, es0, es1, gs0, gs1, ss0, ss1):
    cid = lax.axis_index("c")
    tid = lax.axis_index("s")
    sidx = [si0, si1, si2, si3]
    didx = [di0, di1, di2, di3]
    ebuf = [eb0, eb1]
    gbuf = [gb0, gb1]
    xsem = [xs0, xs1, xs2, xs3]
    esem = [es0, es1]
    gsem = [gs0, gs1]
    ssem = [ss0, ss1]

    for cc in range(NCHUNK // 2):
        chunk = cid * (NCHUNK // 2) + cc
        pltpu.sync_copy(zeros_hbm.at[pl.ds(tid * RPT, RPT)],
                        acc.at[pl.ds(tid * RPT, RPT)])

        @pl.when(tid == 0)
        def _():
            pltpu.sync_copy(zeros_hbm.at[pl.ds(RTB, RTAIL)],
                            acc.at[pl.ds(RTB, RTAIL)])
        plsc.subcore_barrier()
        ebase = chunk * E + tid * EPT
        dbase = tid * EPT

        def start_idx(jj, r, pb):
            off = jj * EB
            pltpu.async_copy(srcadj_hbm.at[pl.ds(ebase + off, EB)],
                             sidx[r], xsem[r])
            pltpu.async_copy(dst_hbm.at[pl.ds(dbase + off, EB)],
                             didx[r], xsem[r])
            pltpu.async_copy(em_hbm.at[pl.ds(ebase + off, EB)],
                             ebuf[pb], esem[pb])

        def wait_idx(r, pb):
            pltpu.make_async_copy(srcadj_hbm.at[pl.ds(ebase, EB)],
                                  sidx[r], xsem[r]).wait()
            pltpu.make_async_copy(dst_hbm.at[pl.ds(dbase, EB)],
                                  didx[r], xsem[r]).wait()
            pltpu.make_async_copy(em_hbm.at[pl.ds(ebase, EB)],
                                  ebuf[pb], esem[pb]).wait()

        def start_gather(r, pb):
            pltpu.async_copy(hm_hbm.at[sidx[r]], gbuf[pb], gsem[pb])

        def wait_gather(r, pb):
            pltpu.make_async_copy(hm_hbm.at[sidx[r]], gbuf[pb],
                                  gsem[pb]).wait()

        def start_scatter(r, pb):
            pltpu.async_copy(gbuf[pb], acc.at[didx[r]], ssem[pb], add=True)

        def wait_scatter(r, pb):
            pltpu.make_async_copy(gbuf[pb], acc.at[didx[r]],
                                  ssem[pb]).wait()

        def compute(pb):
            g = gbuf[pb]
            e = ebuf[pb]

            @plsc.parallel_loop(0, EB, 1, unroll=2)
            def _(row):
                for kk in range(CW // 16):
                    sl = pl.ds(kk * 16, 16)
                    g[row, sl] = jnp.maximum(g[row, sl] + e[row, sl], 0.0)

        def emit_block(jj, r, pb, first=False, last=False):
            nr = (r + 1) % 4
            npb = 1 - pb
            if first:
                pass
            else:
                wait_scatter((r - 1) % 4, npb)
            wait_gather(r, pb)
            compute(pb)
            start_scatter(r, pb)
            if not last:
                wait_idx(nr, npb)
                start_gather(nr, npb)

                @pl.when(jj + 2 < NJ)
                def _():
                    start_idx(jj + 2, (r + 2) % 4, pb)

        start_idx(jnp.int32(0), 0, 0)
        start_idx(jnp.int32(1), 1, 1)
        wait_idx(0, 0)
        start_gather(0, 0)

        emit_block(jnp.int32(0), 0, 0, first=True)
        emit_block(jnp.int32(1), 1, 1)
        emit_block(jnp.int32(2), 2, 0)
        emit_block(jnp.int32(3), 3, 1)

        def quad2(p, _):
            jj = 4 * p
            emit_block(jj, 0, 0)
            emit_block(jj + 1, 1, 1)
            emit_block(jj + 2, 2, 0)
            emit_block(jj + 3, 3, 1)
            return 0
        lax.fori_loop(1, NJ // 4, quad2, 0)
        emit_block(jnp.int32(NJ - 1), 0, 0, last=True)
        wait_scatter(0, 0)

        plsc.subcore_barrier()
        pltpu.sync_copy(acc.at[pl.ds(tid * RPT, RPT)],
                        out_hbm.at[pl.ds(chunk * N + tid * RPT, RPT)])

        @pl.when(tid == 0)
        def _():
            pltpu.sync_copy(acc.at[pl.ds(RTB, RTAIL)],
                            out_hbm.at[pl.ds(chunk * N + RTB, RTAIL)])


def kernel(x, edge_index, edge_attr, batch, params):
    p = params
    src = edge_index[0]
    dst = edge_index[1]
    batch2 = batch[:, None]

    zeros_cw = jnp.zeros((N, CW), jnp.float32)
    zeros_16 = jnp.zeros((N, 16), jnp.float32)
    srcadj = (src[None, :]
              + (N * jnp.arange(NCHUNK, dtype=jnp.int32))[:, None]).reshape(-1)

    h = _linear(x, p['W_in'], p['b_in'][None], 1000)
    deg = _deg_kernel(dst, zeros_16)[:, 0:1]

    for i in range(L):
        wm1 = p['Wm1'][i]
        wm1h = wm1[:HID].reshape(HID, NCHUNK, CW).transpose(1, 0, 2)
        wm1e = wm1[HID:].reshape(EDGE_DIM, NCHUNK, CW).transpose(1, 0, 2)
        bm1 = p['bm1'][i].reshape(NCHUNK, 1, CW)
        zero_b = jnp.zeros((NCHUNK, 1, CW), jnp.float32)
        hm = _chunked_matmul(h, wm1h, bm1, 1000)
        em = _chunked_matmul(edge_attr, wm1e, zero_b, 2000)
        s = _edge_kernel(hm.reshape(NCHUNK * N, CW), srcadj, dst,
                         em.reshape(NCHUNK * E, CW), zeros_cw)
        s4 = s.reshape(NCHUNK, N, CW)
        h = _update(h, s4, deg,
                    p['Wm2'][i].reshape(NCHUNK, CW, HID), p['bm2'][i][None],
                    p['Wu'][i][:HID], p['Wu'][i][HID:], p['bu'][i][None],
                    p['norm_w'][i][None],
                    p['Wf1'][i], p['bf1'][i][None],
                    p['Wf2'][i], p['bf2'][i][None], 1000)

    z, att, m_nodes, cnt = _pool1(h, batch2, p['Wg'], p['bg'][None],
                                  p['Wa'], p['ba'][None], 1000)
    mfin, pad, apad = _poolm(m_nodes, cnt, p['bg'][None], p['Wa'], p['ba'][None])
    sz, s1 = _pool2(z, att, batch2, mfin, 1000)
    return _pool3(sz, s1.T, mfin.T, pad.T, apad,
                  p['Wp1'], p['bp1'][None], p['pnorm_w'][None],
                  p['Wp2'], p['bp2'][None])


# speedup vs baseline: 3.5378x; 1.1039x over previous
import functools

import jax
import jax.numpy as jnp
from jax import lax
from jax.experimental import pallas as pl
from jax.experimental.pallas import tpu as pltpu
from jax.experimental.pallas import tpu_sc as plsc

N = 10000
E = 160000
IN_DIM = 177
EDGE_DIM = 30
HID = 512
PROJ = 768
L = 6
B = 64
EPS = 1.1920928955078125e-07

NCHUNK = 4
CW = HID // NCHUNK
NTILE = 16
RPT = 624
RTAIL = N - NTILE * RPT
RTB = NTILE * RPT
EB = 80
EPT = E // NTILE
NJ = EPT // EB

_mesh = plsc.VectorSubcoreMesh(core_axis_name="c", subcore_axis_name="s",
                               num_cores=2, num_subcores=NTILE)


def _lin_body(x_ref, w_ref, b_ref, o_ref):
    o_ref[...] = (
        jnp.dot(x_ref[...], w_ref[...], preferred_element_type=jnp.float32)
        + b_ref[...]
    )


def _linear(x, w, b, bm):
    m, k = x.shape
    n = w.shape[1]
    return pl.pallas_call(
        _lin_body,
        grid=(m // bm,),
        in_specs=[
            pl.BlockSpec((bm, k), lambda i: (i, 0)),
            pl.BlockSpec((k, n), lambda i: (0, 0)),
            pl.BlockSpec((1, n), lambda i: (0, 0)),
        ],
        out_specs=pl.BlockSpec((bm, n), lambda i: (i, 0)),
        out_shape=jax.ShapeDtypeStruct((m, n), jnp.float32),
    )(x, w, b)


def _chunked_body(x_ref, w_ref, b_ref, o_ref):
    o_ref[...] = (
        jnp.dot(x_ref[...], w_ref[0], preferred_element_type=jnp.float32)
        + b_ref[0]
    )[None]


def _chunked_matmul(x, w, b, bm):
    m, k = x.shape
    return pl.pallas_call(
        _chunked_body,
        grid=(NCHUNK, m // bm),
        in_specs=[
            pl.BlockSpec((bm, k), lambda c, i: (i, 0)),
            pl.BlockSpec((1, k, CW), lambda c, i: (c, 0, 0)),
            pl.BlockSpec((1, 1, CW), lambda c, i: (c, 0, 0)),
        ],
        out_specs=pl.BlockSpec((1, bm, CW), lambda c, i: (c, i, 0)),
        out_shape=jax.ShapeDtypeStruct((NCHUNK, m, CW), jnp.float32),
    )(x, w, b)


def _rms(v, w):
    return v * lax.rsqrt(jnp.mean(v * v, axis=-1, keepdims=True) + EPS) * w


def _upd_body(h_ref, s_ref, deg_ref, wm2_ref, bm2_ref, wuh_ref, wua_ref,
              bu_ref, nw_ref, wf1_ref, bf1_ref, wf2_ref, bf2_ref, o_ref):
    h = h_ref[...]
    t = jnp.dot(s_ref[0], wm2_ref[0], preferred_element_type=jnp.float32)
    for c in range(1, NCHUNK):
        t = t + jnp.dot(s_ref[c], wm2_ref[c], preferred_element_type=jnp.float32)
    aggr = t + deg_ref[...] * bm2_ref[...]
    u = jnp.maximum(
        jnp.dot(h, wuh_ref[...], preferred_element_type=jnp.float32)
        + jnp.dot(aggr, wua_ref[...], preferred_element_type=jnp.float32)
        + bu_ref[...], 0.0)
    h1 = _rms(h + u, nw_ref[...])
    f = jnp.dot(
        jnp.maximum(jnp.dot(h1, wf1_ref[...], preferred_element_type=jnp.float32)
                    + bf1_ref[...], 0.0),
        wf2_ref[...], preferred_element_type=jnp.float32) + bf2_ref[...]
    o_ref[...] = _rms(h1 + f, nw_ref[...])


def _update(h, s4, deg, wm2c, bm2, wuh, wua, bu, nw, wf1, bf1, wf2, bf2, bm):
    full = lambda shape: pl.BlockSpec(shape, lambda i: tuple(0 for _ in shape))
    return pl.pallas_call(
        _upd_body,
        grid=(N // bm,),
        in_specs=[
            pl.BlockSpec((bm, HID), lambda i: (i, 0)),
            pl.BlockSpec((NCHUNK, bm, CW), lambda i: (0, i, 0)),
            pl.BlockSpec((bm, 1), lambda i: (i, 0)),
            full((NCHUNK, CW, HID)),
            full((1, HID)),
            full((HID, HID)),
            full((HID, HID)),
            full((1, HID)),
            full((1, HID)),
            full((HID, HID)),
            full((1, HID)),
            full((HID, HID)),
            full((1, HID)),
        ],
        out_specs=pl.BlockSpec((bm, HID), lambda i: (i, 0)),
        out_shape=jax.ShapeDtypeStruct((N, HID), jnp.float32),
    )(h, s4, deg, wm2c, bm2, wuh, wua, bu, nw, wf1, bf1, wf2, bf2)


def _pool1_body(h_ref, batch_ref, wg_ref, bg_ref, wa_ref, ba_ref,
                z_ref, att_ref, m_ref, cnt_ref):
    z = jnp.dot(h_ref[...], wg_ref[...], preferred_element_type=jnp.float32) + bg_ref[...]
    att = jnp.dot(z, wa_ref[...], preferred_element_type=jnp.float32) + ba_ref[...]
    z_ref[...] = z
    att_ref[...] = att
    iota = lax.broadcasted_iota(jnp.int32, (z.shape[0], B), 1)
    oh = batch_ref[...] == iota
    mloc = jnp.max(jnp.where(oh, att, -jnp.inf), axis=0, keepdims=True)
    cloc = jnp.sum(oh.astype(jnp.float32), axis=0, keepdims=True)

    @pl.when(pl.program_id(0) == 0)
    def _():
        m_ref[...] = jnp.full((1, B), -jnp.inf, jnp.float32)
        cnt_ref[...] = jnp.zeros((1, B), jnp.float32)

    m_ref[...] = jnp.maximum(m_ref[...], mloc)
    cnt_ref[...] = cnt_ref[...] + cloc


def _pool1(h, batch2, wg, bg, wa, ba, bm):
    full = lambda shape: pl.BlockSpec(shape, lambda i: tuple(0 for _ in shape))
    return pl.pallas_call(
        _pool1_body,
        grid=(N // bm,),
        in_specs=[
            pl.BlockSpec((bm, HID), lambda i: (i, 0)),
            pl.BlockSpec((bm, 1), lambda i: (i, 0)),
            full((HID, HID)),
            full((1, HID)),
            full((HID, 1)),
            full((1, 1)),
        ],
        out_specs=[
            pl.BlockSpec((bm, HID), lambda i: (i, 0)),
            pl.BlockSpec((bm, 1), lambda i: (i, 0)),
            full((1, B)),
            full((1, B)),
        ],
        out_shape=[
            jax.ShapeDtypeStruct((N, HID), jnp.float32),
            jax.ShapeDtypeStruct((N, 1), jnp.float32),
            jax.ShapeDtypeStruct((1, B), jnp.float32),
            jax.ShapeDtypeStruct((1, B), jnp.float32),
        ],
    )(h, batch2, wg, bg, wa, ba)


def _poolm_body(m_ref, cnt_ref, bg_ref, wa_ref, ba_ref,
                mfin_ref, pad_ref, apad_ref):
    apad = jnp.dot(bg_ref[...], wa_ref[...], preferred_element_type=jnp.float32) + ba_ref[...]
    cnt = cnt_ref[...]
    mmax = jnp.max(cnt)
    pad = mmax - cnt
    mfin_ref[...] = jnp.where(pad > 0, jnp.maximum(m_ref[...], apad), m_ref[...])
    pad_ref[...] = pad
    apad_ref[...] = apad


def _poolm(m_nodes, cnt, bg, wa, ba):
    full = lambda shape: pl.BlockSpec(shape, lambda: tuple(0 for _ in shape))
    return pl.pallas_call(
        _poolm_body,
        in_specs=[full((1, B)), full((1, B)), full((1, HID)), full((HID, 1)),
                  full((1, 1))],
        out_specs=[full((1, B)), full((1, B)), full((1, 1))],
        out_shape=[
            jax.ShapeDtypeStruct((1, B), jnp.float32),
            jax.ShapeDtypeStruct((1, B), jnp.float32),
            jax.ShapeDtypeStruct((1, 1), jnp.float32),
        ],
    )(m_nodes, cnt, bg, wa, ba)


def _pool2_body(z_ref, att_ref, batch_ref, mfin_ref, sz_ref, s1_ref):
    bm = z_ref.shape[0]
    iota = lax.broadcasted_iota(jnp.int32, (bm, B), 1)
    oh = (batch_ref[...] == iota).astype(jnp.float32)
    msel = jnp.sum(oh * mfin_ref[...], axis=1, keepdims=True)
    w = jnp.exp(att_ref[...] - msel)
    ow = oh * w
    szloc = lax.dot_general(ow, z_ref[...], (((0,), (0,)), ((), ())),
                            preferred_element_type=jnp.float32)
    s1loc = jnp.sum(ow, axis=0, keepdims=True)

    @pl.when(pl.program_id(0) == 0)
    def _():
        sz_ref[...] = jnp.zeros_like(sz_ref)
        s1_ref[...] = jnp.zeros_like(s1_ref)

    sz_ref[...] = sz_ref[...] + szloc
    s1_ref[...] = s1_ref[...] + s1loc


def _pool2(z, att, batch2, mfin, bm):
    full = lambda shape: pl.BlockSpec(shape, lambda i: tuple(0 for _ in shape))
    return pl.pallas_call(
        _pool2_body,
        grid=(N // bm,),
        in_specs=[
            pl.BlockSpec((bm, HID), lambda i: (i, 0)),
            pl.BlockSpec((bm, 1), lambda i: (i, 0)),
            pl.BlockSpec((bm, 1), lambda i: (i, 0)),
            full((1, B)),
        ],
        out_specs=[full((B, HID)), full((1, B))],
        out_shape=[
            jax.ShapeDtypeStruct((B, HID), jnp.float32),
            jax.ShapeDtypeStruct((1, B), jnp.float32),
        ],
    )(z, att, batch2, mfin)


def _pool3_body(sz_ref, s1_ref, mfin_ref, pad_ref, apad_ref,
                wp1_ref, bp1_ref, pnw_ref, wp2_ref, bp2_ref, o_ref):
    denom = s1_ref[...] + pad_ref[...] * jnp.exp(apad_ref[...] - mfin_ref[...])
    hg = sz_ref[...] / denom
    p = jnp.dot(hg, wp1_ref[...], preferred_element_type=jnp.float32) + bp1_ref[...]
    p = jnp.maximum(_rms(p, pnw_ref[...]), 0.0)
    o_ref[...] = jnp.dot(p, wp2_ref[...], preferred_element_type=jnp.float32) + bp2_ref[...]


def _pool3(sz, s1c, mfinc, padc, apad, wp1, bp1, pnw, wp2, bp2):
    full = lambda shape: pl.BlockSpec(shape, lambda: tuple(0 for _ in shape))
    return pl.pallas_call(
        _pool3_body,
        in_specs=[full((B, HID)), full((B, 1)), full((B, 1)), full((B, 1)),
                  full((1, 1)), full((HID, PROJ)), full((1, PROJ)),
                  full((1, PROJ)), full((PROJ, PROJ)), full((1, PROJ))],
        out_specs=full((B, PROJ)),
        out_shape=jax.ShapeDtypeStruct((B, PROJ), jnp.float32),
    )(sz, s1c, mfinc, padc, apad, wp1, bp1, pnw, wp2, bp2)


@functools.partial(
    pl.kernel,
    out_type=jax.ShapeDtypeStruct((N, 16), jnp.float32),
    mesh=_mesh,
    scratch_types=[
        pltpu.VMEM((EB,), jnp.int32),
        pltpu.VMEM((EB, 16), jnp.float32),
        pltpu.VMEM_SHARED((N, 16), jnp.float32),
    ],
)
def _deg_kernel(dst_hbm, zeros_hbm, out_hbm, idx_v, ones_v, acc):
    cid = lax.axis_index("c")
    tid = lax.axis_index("s")

    @pl.when(cid == 0)
    def _():
        def fill(i, _):
            ones_v[i, :] = jnp.full((16,), 1.0, jnp.float32)
            return 0
        lax.fori_loop(0, EB, fill, 0)
        pltpu.sync_copy(zeros_hbm.at[pl.ds(tid * RPT, RPT)],
                        acc.at[pl.ds(tid * RPT, RPT)])

        @pl.when(tid == 0)
        def _():
            pltpu.sync_copy(zeros_hbm.at[pl.ds(RTB, RTAIL)],
                            acc.at[pl.ds(RTB, RTAIL)])
        plsc.subcore_barrier()
        base = tid * EPT

        def step(j, _):
            pltpu.sync_copy(dst_hbm.at[pl.ds(base + j * EB, EB)], idx_v)
            pltpu.sync_copy(ones_v, acc.at[idx_v], add=True)
            return 0
        lax.fori_loop(0, NJ, step, 0)
        plsc.subcore_barrier()
        pltpu.sync_copy(acc.at[pl.ds(tid * RPT, RPT)],
                        out_hbm.at[pl.ds(tid * RPT, RPT)])

        @pl.when(tid == 0)
        def _():
            pltpu.sync_copy(acc.at[pl.ds(RTB, RTAIL)],
                            out_hbm.at[pl.ds(RTB, RTAIL)])


@functools.partial(
    pl.kernel,
    out_type=jax.ShapeDtypeStruct((NCHUNK * N, CW), jnp.float32),
    mesh=_mesh,
    scratch_types=[
        pltpu.VMEM((EB,), jnp.int32),
        pltpu.VMEM((EB,), jnp.int32),
        pltpu.VMEM((EB,), jnp.int32),
        pltpu.VMEM((EB,), jnp.int32),
        pltpu.VMEM((EB,), jnp.int32),
        pltpu.VMEM((EB,), jnp.int32),
        pltpu.VMEM((EB,), jnp.int32),
        pltpu.VMEM((EB,), jnp.int32),
        pltpu.VMEM((EB, CW), jnp.float32),
        pltpu.VMEM((EB, CW), jnp.float32),
        pltpu.VMEM((EB, CW), jnp.float32),
        pltpu.VMEM((EB, CW), jnp.float32),
        pltpu.VMEM_SHARED((N, CW), jnp.float32),
        pltpu.SemaphoreType.DMA,
        pltpu.SemaphoreType.DMA,
        pltpu.SemaphoreType.DMA,
        pltpu.SemaphoreType.DMA,
        pltpu.SemaphoreType.DMA,
        pltpu.SemaphoreType.DMA,
        pltpu.SemaphoreType.DMA,
        pltpu.SemaphoreType.DMA,
        pltpu.SemaphoreType.DMA,
        pltpu.SemaphoreType.DMA,
        pltpu.SemaphoreType.DMA,
        pltpu.SemaphoreType.DMA,
        pltpu.SemaphoreType.DMA,
        pltpu.SemaphoreType.DMA,
        pltpu.SemaphoreType.DMA,
        pltpu.SemaphoreType.DMA,
    ],
)
def _edge_kernel(hm_hbm, srcadj_hbm, dst_hbm, em_hbm, zeros_hbm, out_hbm,
                 si0, si1, si2, si3, di0, di1, di2, di3,
                 gb0, gb1, gb2, gb3, acc,
                 xs0, xs1, xs2, xs3, es0, es1, es2, es3,
                 gs0, gs1, gs2, gs3, ss0, ss1, ss2, ss3):
    cid = lax.axis_index("c")
    tid = lax.axis_index("s")
    sidx = [si0, si1, si2, si3]
    didx = [di0, di1, di2, di3]
    gbuf = [gb0, gb1, gb2, gb3]
    xsem = [xs0, xs1, xs2, xs3]
    esem = [es0, es1, es2, es3]
    gsem = [gs0, gs1, gs2, gs3]
    ssem = [ss0, ss1, ss2, ss3]

    for cc in range(NCHUNK // 2):
        chunk = cid * (NCHUNK // 2) + cc
        pltpu.sync_copy(zeros_hbm.at[pl.ds(tid * RPT, RPT)],
                        acc.at[pl.ds(tid * RPT, RPT)])

        @pl.when(tid == 0)
        def _():
            pltpu.sync_copy(zeros_hbm.at[pl.ds(RTB, RTAIL)],
                            acc.at[pl.ds(RTB, RTAIL)])
        plsc.subcore_barrier()
        ebase = chunk * E + tid * EPT
        dbase = tid * EPT

        def start_idx(jj, b):
            off = jj * EB
            pltpu.async_copy(srcadj_hbm.at[pl.ds(ebase + off, EB)],
                             sidx[b], xsem[b])
            pltpu.async_copy(dst_hbm.at[pl.ds(dbase + off, EB)],
                             didx[b], xsem[b])
            pltpu.async_copy(em_hbm.at[pl.ds(ebase + off, EB)],
                             gbuf[b], esem[b])

        def wait_idx(b):
            pltpu.make_async_copy(srcadj_hbm.at[pl.ds(ebase, EB)],
                                  sidx[b], xsem[b]).wait()
            pltpu.make_async_copy(dst_hbm.at[pl.ds(dbase, EB)],
                                  didx[b], xsem[b]).wait()
            pltpu.make_async_copy(em_hbm.at[pl.ds(ebase, EB)],
                                  gbuf[b], esem[b]).wait()

        def start_gather(b):
            pltpu.async_copy(hm_hbm.at[sidx[b]], gbuf[b], gsem[b], add=True)

        def wait_gather(b):
            pltpu.make_async_copy(hm_hbm.at[sidx[b]], gbuf[b],
                                  gsem[b]).wait()

        def start_scatter(b):
            pltpu.async_copy(gbuf[b], acc.at[didx[b]], ssem[b], add=True)

        def wait_scatter(b):
            pltpu.make_async_copy(gbuf[b], acc.at[didx[b]],
                                  ssem[b]).wait()

        def compute(b):
            g = gbuf[b]

            @plsc.parallel_loop(0, EB, 1, unroll=2)
            def _(row):
                for kk in range(CW // 16):
                    sl = pl.ds(kk * 16, 16)
                    g[row, sl] = jnp.maximum(g[row, sl], 0.0)

        def emit_block(jj, b, depth2=True, last=False):
            nb = (b + 1) % 4
            if depth2:
                wait_scatter((b + 2) % 4)
            if not last:
                @pl.when(jj + 2 < NJ)
                def _():
                    start_idx(jj + 2, (b + 2) % 4)
            wait_gather(b)
            compute(b)
            start_scatter(b)
            if not last:
                wait_idx(nb)
                start_gather(nb)

        start_idx(jnp.int32(0), 0)
        start_idx(jnp.int32(1), 1)
        wait_idx(0)
        start_gather(0)

        emit_block(jnp.int32(0), 0, depth2=False)
        emit_block(jnp.int32(1), 1, depth2=False)
        emit_block(jnp.int32(2), 2)
        emit_block(jnp.int32(3), 3)

        def quad2(p, _):
            jj = 4 * p
            emit_block(jj, 0)
            emit_block(jj + 1, 1)
            emit_block(jj + 2, 2)
            emit_block(jj + 3, 3)
            return 0
        lax.fori_loop(1, NJ // 4, quad2, 0)
        emit_block(jnp.int32(NJ - 1), 0, last=True)
        wait_scatter(3)
        wait_scatter(0)

        plsc.subcore_barrier()
        pltpu.sync_copy(acc.at[pl.ds(tid * RPT, RPT)],
                        out_hbm.at[pl.ds(chunk * N + tid * RPT, RPT)])

        @pl.when(tid == 0)
        def _():
            pltpu.sync_copy(acc.at[pl.ds(RTB, RTAIL)],
                            out_hbm.at[pl.ds(chunk * N + RTB, RTAIL)])


def kernel(x, edge_index, edge_attr, batch, params):
    p = params
    src = edge_index[0]
    dst = edge_index[1]
    batch2 = batch[:, None]

    zeros_cw = jnp.zeros((N, CW), jnp.float32)
    zeros_16 = jnp.zeros((N, 16), jnp.float32)
    srcadj = (src[None, :]
              + (N * jnp.arange(NCHUNK, dtype=jnp.int32))[:, None]).reshape(-1)

    h = _linear(x, p['W_in'], p['b_in'][None], 1000)
    deg = _deg_kernel(dst, zeros_16)[:, 0:1]

    for i in range(L):
        wm1 = p['Wm1'][i]
        wm1h = wm1[:HID].reshape(HID, NCHUNK, CW).transpose(1, 0, 2)
        wm1e = wm1[HID:].reshape(EDGE_DIM, NCHUNK, CW).transpose(1, 0, 2)
        bm1 = p['bm1'][i].reshape(NCHUNK, 1, CW)
        zero_b = jnp.zeros((NCHUNK, 1, CW), jnp.float32)
        hm = _chunked_matmul(h, wm1h, bm1, 1000)
        em = _chunked_matmul(edge_attr, wm1e, zero_b, 2000)
        s = _edge_kernel(hm.reshape(NCHUNK * N, CW), srcadj, dst,
                         em.reshape(NCHUNK * E, CW), zeros_cw)
        s4 = s.reshape(NCHUNK, N, CW)
        h = _update(h, s4, deg,
                    p['Wm2'][i].reshape(NCHUNK, CW, HID), p['bm2'][i][None],
                    p['Wu'][i][:HID], p['Wu'][i][HID:], p['bu'][i][None],
                    p['norm_w'][i][None],
                    p['Wf1'][i], p['bf1'][i][None],
                    p['Wf2'][i], p['bf2'][i][None], 1000)

    z, att, m_nodes, cnt = _pool1(h, batch2, p['Wg'], p['bg'][None],
                                  p['Wa'], p['ba'][None], 1000)
    mfin, pad, apad = _poolm(m_nodes, cnt, p['bg'][None], p['Wa'], p['ba'][None])
    sz, s1 = _pool2(z, att, batch2, mfin, 1000)
    return _pool3(sz, s1.T, mfin.T, pad.T, apad,
                  p['Wp1'], p['bp1'][None], p['pnorm_w'][None],
                  p['Wp2'], p['bp2'][None])
